# baseline (speedup 1.0000x reference)
import jax
import jax.numpy as jnp
from jax import lax
from jax.experimental import pallas as pl
from jax.experimental.pallas import tpu as pltpu

S, D, H, E, TOPK, F = 2048, 1024, 64, 16, 2, 64

_TT = (((0,), (0,)), ((), ()))


def _halving_sum(p):
    m = p.shape[0]
    while m > 8:
        h = m // 2
        p = p[:h] + p[h:]
        m = h
    return jnp.sum(p, axis=0, keepdims=True)


def _moe_body(x_ref, wout_ref, win_ref, wlin_ref, we_hbm, o_ref,
              ws_ref, sem0, sem1):
    x = x_ref[...]
    vcol = lax.dot_general(x, wout_ref[...], _TT,
                           preferred_element_type=jnp.float32)

    t = _halving_sum(win_ref[...] * vcol)
    tcol = lax.transpose(t, (1, 0))
    g = _halving_sum(wlin_ref[...] * tcol)

    iota = lax.broadcasted_iota(jnp.int32, (1, E), 1)
    m1 = jnp.max(g)
    i1 = jnp.min(jnp.where(g == m1, iota, E))
    g2 = jnp.where(iota == i1, -jnp.inf, g)
    m2 = jnp.max(g2)
    i2 = jnp.min(jnp.where(g2 == m2, iota, E))
    w1 = 1.0 / (1.0 + jnp.exp(m2 - m1))
    w2 = 1.0 - w1

    cp0 = pltpu.make_async_copy(we_hbm.at[pl.ds(i1, 1)],
                                ws_ref.at[pl.ds(0, 1)], sem0)
    cp1 = pltpu.make_async_copy(we_hbm.at[pl.ds(i2, 1)],
                                ws_ref.at[pl.ds(1, 1)], sem1)
    cp0.start()
    cp1.start()
    cp0.wait()
    cp1.wait()

    ones_col = jnp.full((F, 1), 1.0, dtype=jnp.float32)
    c = jnp.float32(0.7071067811865476)

    def expert(Wk, wk):
        zk = jnp.dot(x, Wk, preferred_element_type=jnp.float32)
        nn = lax.dot_general(zk * zk, ones_col,
                             (((1,), (0,)), ((), ())),
                             preferred_element_type=jnp.float32)
        inv = 1.0 / jnp.maximum(jnp.sqrt(nn), 1e-12)
        zn = zk * inv
        return (0.5 * wk) * (zn * (1.0 + lax.erf(zn * c)))

    o_ref[...] = expert(ws_ref[0], w1) + expert(ws_ref[1], w2)


def kernel(x, W_gate_in, W_gate_lin, W_gate_out, W_experts):
    return pl.pallas_call(
        _moe_body,
        in_specs=[
            pl.BlockSpec((S, D), lambda: (0, 0)),
            pl.BlockSpec((S, 1), lambda: (0, 0)),
            pl.BlockSpec((D, H), lambda: (0, 0)),
            pl.BlockSpec((H, E), lambda: (0, 0)),
            pl.BlockSpec(memory_space=pl.ANY),
        ],
        out_specs=pl.BlockSpec((S, F), lambda: (0, 0)),
        out_shape=jax.ShapeDtypeStruct((S, F), jnp.float32),
        scratch_shapes=[
            pltpu.VMEM((TOPK, D, F), jnp.float32),
            pltpu.SemaphoreType.DMA,
            pltpu.SemaphoreType.DMA,
        ],
    )(x, W_gate_out, W_gate_in, W_gate_lin, W_experts)
